# Initial kernel scaffold; baseline (speedup 1.0000x reference)
import jax
import jax.numpy as jnp
from jax.experimental import pallas as pl


def kernel(x, edge_index, edge_weights, batch, graph_features, W_gat, a_src, a_dst, b_gat, W2, b2, W3, b3, Wg1, bg1, Wg2, bg2, Wfc1, bfc1, Wfc2, bfc2, bn1_g, bn1_b, bn1_m, bn1_v, bn2_g, bn2_b, bn2_m, bn2_v):
    raise NotImplementedError("write your pallas kernel here")


# speedup vs baseline: 2.7512x; 2.7512x over previous
import functools

import jax
import jax.numpy as jnp
from jax.experimental import pallas as pl

N = 10000
B = 64
H = 8
C = 32


def _mm_body(x_ref, w_ref, si_ref, ti_ref, so_ref, to_ref, o_ref, *,
             relu_in, relu_out):
    x = x_ref[...]
    x = x * si_ref[...] + ti_ref[...]
    if relu_in:
        x = jnp.maximum(x, 0.0)
    y = jnp.dot(x, w_ref[...], preferred_element_type=jnp.float32)
    y = y * so_ref[...] + to_ref[...]
    if relu_out:
        y = jnp.maximum(y, 0.0)
    o_ref[...] = y


def _mm(x, W, scale_in=None, shift_in=None, relu_in=False,
        scale_out=None, shift_out=None, relu_out=False, bm=1024):
    M, K = x.shape
    Nout = W.shape[1]
    one_k = jnp.ones((1, K), jnp.float32)
    zero_k = jnp.zeros((1, K), jnp.float32)
    one_n = jnp.ones((1, Nout), jnp.float32)
    zero_n = jnp.zeros((1, Nout), jnp.float32)
    si = one_k if scale_in is None else scale_in.reshape(1, K)
    ti = zero_k if shift_in is None else shift_in.reshape(1, K)
    so = one_n if scale_out is None else scale_out.reshape(1, Nout)
    to = zero_n if shift_out is None else shift_out.reshape(1, Nout)
    bm = min(bm, M)
    grid = (pl.cdiv(M, bm),)
    return pl.pallas_call(
        functools.partial(_mm_body, relu_in=relu_in, relu_out=relu_out),
        grid=grid,
        in_specs=[
            pl.BlockSpec((bm, K), lambda i: (i, 0)),
            pl.BlockSpec((K, Nout), lambda i: (0, 0)),
            pl.BlockSpec((1, K), lambda i: (0, 0)),
            pl.BlockSpec((1, K), lambda i: (0, 0)),
            pl.BlockSpec((1, Nout), lambda i: (0, 0)),
            pl.BlockSpec((1, Nout), lambda i: (0, 0)),
        ],
        out_specs=pl.BlockSpec((bm, Nout), lambda i: (i, 0)),
        out_shape=jax.ShapeDtypeStruct((M, Nout), jnp.float32),
    )(x, W, si, ti, so, to)


def kernel(x, edge_index, edge_weights, batch, graph_features,
           W_gat, a_src, a_dst, b_gat, W2, b2, W3, b3,
           Wg1, bg1, Wg2, bg2, Wfc1, bfc1, Wfc2, bfc2,
           bn1_g, bn1_b, bn1_m, bn1_v, bn2_g, bn2_b, bn2_m, bn2_v):
    loop = jnp.arange(N, dtype=edge_index.dtype)
    src = jnp.concatenate([edge_index[0], loop])
    dst = jnp.concatenate([edge_index[1], loop])
    w_sl = jnp.concatenate([edge_weights, jnp.ones((N,), jnp.float32)])

    gf = graph_features.reshape(-1, 2048)
    g = _mm(gf, Wg1, shift_out=bg1, relu_out=True)
    g = _mm(g, Wg2, shift_out=bg2)

    xp = _mm(x, W_gat)
    eye = jnp.eye(H, dtype=jnp.float32)
    A_s = (a_src[:, None, :, None] * eye[:, :, None, None]
           ).transpose(0, 2, 1, 3).reshape(H * C, H)
    A_d = (a_dst[:, None, :, None] * eye[:, :, None, None]
           ).transpose(0, 2, 1, 3).reshape(H * C, H)
    al = _mm(xp, jnp.concatenate([A_s, A_d], axis=1))
    al_s, al_d = al[:, :H], al[:, H:]

    e = jax.nn.leaky_relu(al_s[src] + al_d[dst], 0.2)
    m = jax.ops.segment_max(e, dst, num_segments=N)
    m = jnp.where(jnp.isfinite(m), m, 0.0)
    ex = jnp.exp(e - m[dst])
    den = jax.ops.segment_sum(ex, dst, num_segments=N)
    alpha = ex / (den[dst] + 1e-16)
    msg = xp[src] * jnp.repeat(alpha, C, axis=1)
    h1 = jax.ops.segment_sum(msg, dst, num_segments=N)

    deg = jax.ops.segment_sum(w_sl, dst, num_segments=N)
    dinv = jnp.where(deg > 0, jax.lax.rsqrt(jnp.maximum(deg, 1e-12)), 0.0)
    norm = dinv[src] * w_sl * dinv[dst]

    xw2 = _mm(h1, W2, shift_in=b_gat, relu_in=True)
    s2 = jax.ops.segment_sum(xw2[src] * norm[:, None], dst, num_segments=N)
    rs1 = jax.lax.rsqrt(bn1_v + 1e-5)
    sc1 = rs1 * bn1_g
    sh1 = (b2 - bn1_m) * sc1 + bn1_b

    xw3 = _mm(s2, W3, scale_in=sc1, shift_in=sh1, relu_in=True)
    s3 = jax.ops.segment_sum(xw3[src] * norm[:, None], dst, num_segments=N)
    rs2 = jax.lax.rsqrt(bn2_v + 1e-5)
    sc2 = rs2 * bn2_g
    sh2 = (b3 - bn2_m) * sc2 + bn2_b
    h3 = jnp.maximum(s3 * sc2 + sh2, 0.0)

    pooled = jax.ops.segment_max(h3, batch, num_segments=B)
    pooled = jnp.where(jnp.isfinite(pooled), pooled, 0.0)

    z = pooled + g
    z = _mm(z, Wfc1, shift_out=bfc1, relu_out=True)
    return _mm(z, Wfc2, shift_out=bfc2)


# speedup vs baseline: 3.1151x; 1.1323x over previous
import functools

import jax
import jax.numpy as jnp
from jax.experimental import pallas as pl

N = 10000
B = 64
H = 8
C = 32


def _mm_body(x_ref, w_ref, si_ref, ti_ref, so_ref, to_ref, o_ref, *,
             relu_in, relu_out):
    x = x_ref[...]
    x = x * si_ref[...] + ti_ref[...]
    if relu_in:
        x = jnp.maximum(x, 0.0)
    y = jnp.dot(x, w_ref[...], preferred_element_type=jnp.float32)
    y = y * so_ref[...] + to_ref[...]
    if relu_out:
        y = jnp.maximum(y, 0.0)
    o_ref[...] = y


def _mm(x, W, scale_in=None, shift_in=None, relu_in=False,
        scale_out=None, shift_out=None, relu_out=False, bm=1024):
    M, K = x.shape
    Nout = W.shape[1]
    one_k = jnp.ones((1, K), jnp.float32)
    zero_k = jnp.zeros((1, K), jnp.float32)
    one_n = jnp.ones((1, Nout), jnp.float32)
    zero_n = jnp.zeros((1, Nout), jnp.float32)
    si = one_k if scale_in is None else scale_in.reshape(1, K)
    ti = zero_k if shift_in is None else shift_in.reshape(1, K)
    so = one_n if scale_out is None else scale_out.reshape(1, Nout)
    to = zero_n if shift_out is None else shift_out.reshape(1, Nout)
    bm = min(bm, M)
    grid = (pl.cdiv(M, bm),)
    return pl.pallas_call(
        functools.partial(_mm_body, relu_in=relu_in, relu_out=relu_out),
        grid=grid,
        in_specs=[
            pl.BlockSpec((bm, K), lambda i: (i, 0)),
            pl.BlockSpec((K, Nout), lambda i: (0, 0)),
            pl.BlockSpec((1, K), lambda i: (0, 0)),
            pl.BlockSpec((1, K), lambda i: (0, 0)),
            pl.BlockSpec((1, Nout), lambda i: (0, 0)),
            pl.BlockSpec((1, Nout), lambda i: (0, 0)),
        ],
        out_specs=pl.BlockSpec((bm, Nout), lambda i: (i, 0)),
        out_shape=jax.ShapeDtypeStruct((M, Nout), jnp.float32),
    )(x, W, si, ti, so, to)


def kernel(x, edge_index, edge_weights, batch, graph_features,
           W_gat, a_src, a_dst, b_gat, W2, b2, W3, b3,
           Wg1, bg1, Wg2, bg2, Wfc1, bfc1, Wfc2, bfc2,
           bn1_g, bn1_b, bn1_m, bn1_v, bn2_g, bn2_b, bn2_m, bn2_v):
    loop = jnp.arange(N, dtype=edge_index.dtype)
    src = jnp.concatenate([edge_index[0], loop])
    dst = jnp.concatenate([edge_index[1], loop])
    w_sl = jnp.concatenate([edge_weights, jnp.ones((N,), jnp.float32)])

    gf = graph_features.reshape(-1, 2048)
    g = _mm(gf, Wg1, shift_out=bg1, relu_out=True)
    g = _mm(g, Wg2, shift_out=bg2)

    xp = _mm(x, W_gat)
    eye = jnp.eye(H, dtype=jnp.float32)
    A_s = (a_src[:, None, :, None] * eye[:, :, None, None]
           ).transpose(0, 2, 1, 3).reshape(H * C, H)
    A_d = (a_dst[:, None, :, None] * eye[:, :, None, None]
           ).transpose(0, 2, 1, 3).reshape(H * C, H)
    al = _mm(xp, jnp.concatenate([A_s, A_d], axis=1))
    al_s, al_d = al[:, :H], al[:, H:]

    e = jax.nn.leaky_relu(al_s[src] + al_d[dst], 0.2)
    m = jax.ops.segment_max(e, dst, num_segments=N)
    m = jnp.where(jnp.isfinite(m), m, 0.0)
    ex = jnp.exp(e - m[dst])
    den = jax.ops.segment_sum(ex, dst, num_segments=N)
    alpha = ex / (den[dst] + 1e-16)
    msg = xp[src] * jnp.repeat(alpha, C, axis=1)
    h1 = jax.ops.segment_sum(msg, dst, num_segments=N)

    deg = jax.ops.segment_sum(w_sl, dst, num_segments=N)
    dinv = jnp.where(deg > 0, jax.lax.rsqrt(jnp.maximum(deg, 1e-12)), 0.0)
    norm = dinv[src] * w_sl * dinv[dst]

    xw2 = _mm(h1, W2, shift_in=b_gat, relu_in=True)
    s2 = jax.ops.segment_sum(xw2[src] * norm[:, None], dst, num_segments=N)
    rs1 = jax.lax.rsqrt(bn1_v + 1e-5)
    sc1 = rs1 * bn1_g
    sh1 = (b2 - bn1_m) * sc1 + bn1_b

    h2 = jnp.maximum(s2 * sc1 + sh1, 0.0)
    a3 = jax.ops.segment_sum(h2[src] * norm[:, None], dst, num_segments=N)
    rs2 = jax.lax.rsqrt(bn2_v + 1e-5)
    sc2 = rs2 * bn2_g
    sh2 = (b3 - bn2_m) * sc2 + bn2_b
    h3 = _mm(a3, W3, scale_out=sc2, shift_out=sh2, relu_out=True)

    pooled = jax.ops.segment_max(h3, batch, num_segments=B)
    pooled = jnp.where(jnp.isfinite(pooled), pooled, 0.0)

    z = pooled + g
    z = _mm(z, Wfc1, shift_out=bfc1, relu_out=True)
    return _mm(z, Wfc2, shift_out=bfc2)
